# baseline (speedup 1.0000x reference)
import functools

import jax
import jax.numpy as jnp
from jax import lax
from jax.experimental import pallas as pl
from jax.experimental.pallas import tpu as pltpu
from jax.experimental.pallas import tpu_sc as plsc

_D = 128
_NB = 2
_NCLS = 5
_C = 128
_NW = 32


def _mm_body(u_ref, p_ref, o_ref):
    u = u_ref[...]
    o_ref[:, 0:_D] = lax.dot_general(
        u, p_ref[0], (((1,), (0,)), ((), ())),
        preferred_element_type=jnp.float32)
    o_ref[:, _D:2 * _D] = lax.dot_general(
        u, p_ref[1], (((1,), (0,)), ((), ())),
        preferred_element_type=jnp.float32)


def _compute_uh(ufeat, P):
    n, d = ufeat.shape
    blk = 1000
    return pl.pallas_call(
        _mm_body,
        grid=(n // blk,),
        in_specs=[
            pl.BlockSpec((blk, d), lambda i: (i, 0)),
            pl.BlockSpec(P.shape, lambda i: (0, 0, 0)),
        ],
        out_specs=pl.BlockSpec((blk, _NB * d), lambda i: (i, 0)),
        out_shape=jax.ShapeDtypeStruct((n, _NB * d), jnp.float32),
    )(ufeat, P)


def _sc_body(n_chunks, uh_hbm, if_hbm, src_hbm, dst_hbm, w_hbm, out_hbm,
             srcv, dstv, uhv, ifv, wv, outv, sem_u, sem_i):
    wid = lax.axis_index("s") * 2 + lax.axis_index("c")
    tile_base = wid * (n_chunks * _C)
    pltpu.sync_copy(w_hbm, wv)
    wrows = [wv[i, :] for i in range(_NB)]
    w = [[wrows[i][c] for c in range(_NCLS)] for i in range(_NB)]
    lanes = lax.iota(jnp.int32, 16)
    col0 = jnp.zeros((16,), jnp.int32)
    zero = jnp.zeros((16,), jnp.float32)

    def chunk(j, _):
        base = pl.multiple_of(tile_base + j * _C, _C)
        pltpu.sync_copy(src_hbm.at[pl.ds(base, _C)], srcv)
        pltpu.sync_copy(dst_hbm.at[pl.ds(base, _C)], dstv)
        cu = pltpu.async_copy(uh_hbm.at[srcv], uhv, sem_u)
        ci = pltpu.async_copy(if_hbm.at[dstv], ifv, sem_i)
        cu.wait()
        ci.wait()

        ngr = _C // 16
        evecs = [g * 16 + lanes for g in range(ngr)]

        def fstep(f, carry):
            accs = list(carry)
            colv = (lanes + f) & (_D - 1)
            colb = colv + _D
            for g in range(ngr):
                u0 = plsc.load_gather(uhv, [evecs[g], colv])
                u1 = plsc.load_gather(uhv, [evecs[g], colb])
                iv = plsc.load_gather(ifv, [evecs[g], colv])
                accs[2 * g] = accs[2 * g] + u0 * iv
                accs[2 * g + 1] = accs[2 * g + 1] + u1 * iv
            return tuple(accs)

        res = lax.fori_loop(0, _D, fstep, (zero,) * (2 * ngr))
        for g in range(ngr):
            a0 = res[2 * g]
            a1 = res[2 * g + 1]
            for c in range(_NCLS):
                ov = w[0][c] * a0 + w[1][c] * a1
                plsc.store_scatter(
                    outv, [evecs[g], jnp.full((16,), c, jnp.int32)], ov)
        pltpu.sync_copy(outv, out_hbm.at[pl.ds(base, _C), :])
        return 0

    lax.fori_loop(0, n_chunks, chunk, 0)


def _sc_scores(uh, ifeat, src, dst, w2, e_pad, n_chunks):
    mesh = plsc.VectorSubcoreMesh(core_axis_name="c", subcore_axis_name="s")
    f = pl.kernel(
        functools.partial(_sc_body, n_chunks),
        mesh=mesh,
        compiler_params=pltpu.CompilerParams(needs_layout_passes=False),
        out_type=jax.ShapeDtypeStruct((e_pad, _NCLS), jnp.float32),
        scratch_types=[
            pltpu.VMEM((_C,), jnp.int32),
            pltpu.VMEM((_C,), jnp.int32),
            pltpu.VMEM((_C, _NB * _D), jnp.float32),
            pltpu.VMEM((_C, _D), jnp.float32),
            pltpu.VMEM((_NB, 16), jnp.float32),
            pltpu.VMEM((_C, _NCLS), jnp.float32),
            pltpu.SemaphoreType.DMA,
            pltpu.SemaphoreType.DMA,
        ],
    )
    return f(uh, ifeat, src, dst, w2)


def kernel(ufeat, ifeat, edge_index, P, W_combine):
    e = edge_index.shape[1]
    uh = _compute_uh(ufeat, P)
    src = edge_index[0].astype(jnp.int32)
    dst = edge_index[1].astype(jnp.int32)
    n_chunks = -(-e // (_C * _NW))
    e_pad = n_chunks * _C * _NW
    src = jnp.pad(src, (0, e_pad - e))
    dst = jnp.pad(dst, (0, e_pad - e))
    w2 = jnp.zeros((_NB, 16), jnp.float32).at[:, :_NCLS].set(W_combine.T)
    out = _sc_scores(uh, ifeat, src, dst, w2, e_pad, n_chunks)
    return out[:e]


# speedup vs baseline: 1.3189x; 1.3189x over previous
import functools

import jax
import jax.numpy as jnp
from jax import lax
from jax.experimental import pallas as pl
from jax.experimental.pallas import tpu as pltpu
from jax.experimental.pallas import tpu_sc as plsc

_D = 128
_NB = 2
_NCLS = 5
_C = 112
_NW = 32


def _mm_body(u_ref, p_ref, o_ref):
    u = u_ref[...]
    o_ref[:, 0:_D] = lax.dot_general(
        u, p_ref[0], (((1,), (0,)), ((), ())),
        preferred_element_type=jnp.float32)
    o_ref[:, _D:2 * _D] = lax.dot_general(
        u, p_ref[1], (((1,), (0,)), ((), ())),
        preferred_element_type=jnp.float32)


def _compute_uh(ufeat, P):
    n, d = ufeat.shape
    blk = 1000
    return pl.pallas_call(
        _mm_body,
        grid=(n // blk,),
        in_specs=[
            pl.BlockSpec((blk, d), lambda i: (i, 0)),
            pl.BlockSpec(P.shape, lambda i: (0, 0, 0)),
        ],
        out_specs=pl.BlockSpec((blk, _NB * d), lambda i: (i, 0)),
        out_shape=jax.ShapeDtypeStruct((n, _NB * d), jnp.float32),
    )(ufeat, P)


def _sc_body(n_chunks, uh_hbm, if_hbm, src_hbm, dst_hbm, w_hbm, out_hbm,
             srci, dsti, uhv0, uhv1, ifv0, ifv1, wv, outv0, outv1,
             su0, su1, si0, si1, so0, so1):
    wid = lax.axis_index("s") * 2 + lax.axis_index("c")
    uhv = [uhv0, uhv1]
    ifv = [ifv0, ifv1]
    outv = [outv0, outv1]
    su = [su0, su1]
    si = [si0, si1]
    so = [so0, so1]

    rows0 = wid * n_chunks
    pltpu.sync_copy(w_hbm, wv)
    pltpu.sync_copy(src_hbm.at[pl.ds(rows0, n_chunks)], srci)
    pltpu.sync_copy(dst_hbm.at[pl.ds(rows0, n_chunks)], dsti)

    wrows = [wv[i, :] for i in range(_NB)]
    w = [[wrows[i][c] for c in range(_NCLS)] for i in range(_NB)]
    lanes = lax.iota(jnp.int32, 16)
    zero = jnp.zeros((16,), jnp.float32)
    ngr = _C // 16
    evecs = [g * 16 + lanes for g in range(ngr)]
    tile_base = rows0 * _C

    def issue_gathers(k, p):
        pltpu.async_copy(uh_hbm.at[srci.at[k]], uhv[p], su[p])
        pltpu.async_copy(if_hbm.at[dsti.at[k]], ifv[p], si[p])

    def wait_gathers(k, p):
        pltpu.make_async_copy(uh_hbm.at[srci.at[k]], uhv[p], su[p]).wait()
        pltpu.make_async_copy(if_hbm.at[dsti.at[k]], ifv[p], si[p]).wait()

    def out_slice(k):
        base = pl.multiple_of(tile_base + k * _C, _C)
        return out_hbm.at[pl.ds(base, _C), :]

    def wait_out(k, p):
        pltpu.make_async_copy(outv[p], out_slice(k), so[p]).wait()

    def compute(k, p):
        uhr = uhv[p]
        ifr = ifv[p]

        def fstep(f, carry):
            accs = list(carry)
            colv = (lanes + f) & (_D - 1)
            colb = colv + _D
            for g in range(ngr):
                u0 = plsc.load_gather(uhr, [evecs[g], colv])
                u1 = plsc.load_gather(uhr, [evecs[g], colb])
                iv = plsc.load_gather(ifr, [evecs[g], colv])
                accs[2 * g] = accs[2 * g] + u0 * iv
                accs[2 * g + 1] = accs[2 * g + 1] + u1 * iv
            return tuple(accs)

        res = lax.fori_loop(0, _D, fstep, (zero,) * (2 * ngr))
        for g in range(ngr):
            a0 = res[2 * g]
            a1 = res[2 * g + 1]
            for c in range(_NCLS):
                ov = w[0][c] * a0 + w[1][c] * a1
                plsc.store_scatter(
                    outv[p], [evecs[g], jnp.full((16,), c, jnp.int32)], ov)
        pltpu.async_copy(outv[p], out_slice(k), so[p])

    issue_gathers(0, 0)

    def body(jj, _):
        a = 2 * jj
        b = a + 1
        issue_gathers(b, 1)
        wait_gathers(a, 0)

        @pl.when(jj > 0)
        def _():
            wait_out(a - 2, 0)

        compute(a, 0)

        @pl.when(b + 1 < n_chunks)
        def _():
            issue_gathers(b + 1, 0)

        wait_gathers(b, 1)

        @pl.when(jj > 0)
        def _():
            wait_out(b - 2, 1)

        compute(b, 1)
        return 0

    lax.fori_loop(0, n_chunks // 2, body, 0)
    wait_out(n_chunks - 2, 0)
    wait_out(n_chunks - 1, 1)


def _sc_scores(uh, ifeat, src2d, dst2d, w2, e_pad, n_chunks):
    mesh = plsc.VectorSubcoreMesh(core_axis_name="c", subcore_axis_name="s")
    f = pl.kernel(
        functools.partial(_sc_body, n_chunks),
        mesh=mesh,
        compiler_params=pltpu.CompilerParams(
            needs_layout_passes=False, use_tc_tiling_on_sc=False),
        out_type=jax.ShapeDtypeStruct((e_pad, _NCLS), jnp.float32),
        scratch_types=[
            pltpu.VMEM((n_chunks, _C), jnp.int32),
            pltpu.VMEM((n_chunks, _C), jnp.int32),
            pltpu.VMEM((_C, _NB * _D), jnp.float32),
            pltpu.VMEM((_C, _NB * _D), jnp.float32),
            pltpu.VMEM((_C, _D), jnp.float32),
            pltpu.VMEM((_C, _D), jnp.float32),
            pltpu.VMEM((_NB, 16), jnp.float32),
            pltpu.VMEM((_C, _NCLS), jnp.float32),
            pltpu.VMEM((_C, _NCLS), jnp.float32),
            pltpu.SemaphoreType.DMA,
            pltpu.SemaphoreType.DMA,
            pltpu.SemaphoreType.DMA,
            pltpu.SemaphoreType.DMA,
            pltpu.SemaphoreType.DMA,
            pltpu.SemaphoreType.DMA,
        ],
    )
    return f(uh, ifeat, src2d, dst2d, w2)


def kernel(ufeat, ifeat, edge_index, P, W_combine):
    e = edge_index.shape[1]
    uh = _compute_uh(ufeat, P)
    src = edge_index[0].astype(jnp.int32)
    dst = edge_index[1].astype(jnp.int32)
    n_chunks = -(-e // (_C * _NW))
    n_chunks = n_chunks + (n_chunks & 1)
    e_pad = n_chunks * _C * _NW
    src2d = jnp.pad(src, (0, e_pad - e)).reshape(e_pad // _C, _C)
    dst2d = jnp.pad(dst, (0, e_pad - e)).reshape(e_pad // _C, _C)
    w2 = jnp.zeros((_NB, 16), jnp.float32).at[:, :_NCLS].set(W_combine.T)
    out = _sc_scores(uh, ifeat, src2d, dst2d, w2, e_pad, n_chunks)
    return out[:e]
